# baseline (speedup 1.0000x reference)
import functools

import jax
import jax.numpy as jnp
from jax import lax
from jax.experimental import pallas as pl
from jax.experimental.pallas import tpu as pltpu
from jax.experimental.pallas import tpu_sc as plsc

D = 128
N_CAT = 26
D_CONT = D - N_CAT
VOCAB = 1000
EMB_DIM = 32
BATCH = 4096
OUT_D = D_CONT + N_CAT * EMB_DIM

NUM_CORES = 2
NUM_SUBCORES = 16
NW = NUM_CORES * NUM_SUBCORES
ROWS = BATCH // NW
CHUNK = 64
LANES = 16


def _body(x_hbm, tab_hbm, out_hbm, xblk, obuf, gath, idx2d, sem):
    wid = lax.axis_index("s") * NUM_CORES + lax.axis_index("c")
    lanes = lax.iota(jnp.int32, LANES)

    def chunk_pass(c, carry):
        base = wid * ROWS + c * CHUNK

        pltpu.sync_copy(x_hbm.at[pl.ds(base, CHUNK)], xblk)

        def feat(i, cc):
            col = jnp.full((LANES,), i + D_CONT, jnp.int32)
            off = i * VOCAB

            def sub(m, c2):
                rows = m * LANES + lanes
                v = plsc.load_gather(xblk, [rows, col])
                idx2d[i, pl.ds(m * LANES, LANES)] = v.astype(jnp.int32) + off
                return c2

            return lax.fori_loop(0, CHUNK // LANES, sub, cc)

        lax.fori_loop(0, N_CAT, feat, 0)

        def fire(g, cc):
            pltpu.make_async_copy(
                tab_hbm.at[idx2d.at[g]], gath.at[g], sem
            ).start()
            return cc

        lax.fori_loop(0, N_CAT, fire, 0)

        def drain(g, cc):
            pltpu.make_async_copy(
                tab_hbm.at[idx2d.at[g]], gath.at[g], sem
            ).wait()
            return cc

        lax.fori_loop(0, N_CAT, drain, 0)

        def place_row(r, cc):
            rb = r * OUT_D
            def cont_chunk(k, c2):
                v = xblk[r, pl.ds(k * LANES, LANES)]
                obuf[pl.ds(rb + k * LANES, LANES)] = v
                return c2

            lax.fori_loop(0, D_CONT // LANES, cont_chunk, 0)
            v = xblk[r, pl.ds(D_CONT - LANES, LANES)]
            obuf[pl.ds(rb + D_CONT - LANES, LANES)] = v

            def place_feat(g, c2):
                dst = rb + D_CONT + g * EMB_DIM
                lo = gath[g, r, pl.ds(0, LANES)]
                hi = gath[g, r, pl.ds(LANES, LANES)]
                obuf[pl.ds(dst, LANES)] = lo
                obuf[pl.ds(dst + LANES, LANES)] = hi
                return c2

            return lax.fori_loop(0, N_CAT, place_feat, cc)

        lax.fori_loop(0, CHUNK, place_row, 0)

        pltpu.sync_copy(obuf, out_hbm.at[pl.ds(base * OUT_D, CHUNK * OUT_D)])
        return carry

    lax.fori_loop(0, ROWS // CHUNK, chunk_pass, 0)


_sc_call = functools.partial(
    pl.kernel,
    mesh=plsc.VectorSubcoreMesh(core_axis_name="c", subcore_axis_name="s"),
    out_type=jax.ShapeDtypeStruct((BATCH * OUT_D,), jnp.float32),
    scratch_types=[
        pltpu.VMEM((CHUNK, D), jnp.float32),
        pltpu.VMEM((CHUNK * OUT_D,), jnp.float32),
        pltpu.VMEM((N_CAT, CHUNK, EMB_DIM), jnp.float32),
        pltpu.VMEM((N_CAT, CHUNK), jnp.int32),
        pltpu.SemaphoreType.DMA,
    ],
    compiler_params=pltpu.CompilerParams(
        use_tc_tiling_on_sc=False, needs_layout_passes=False
    ),
)(_body)


@jax.jit
def kernel(X, emb_tables, categ_idcs, non_categ_mask):
    tab = emb_tables.reshape(N_CAT * VOCAB, EMB_DIM)
    return _sc_call(X, tab).reshape(BATCH, OUT_D)


# speedup vs baseline: 1.0468x; 1.0468x over previous
import functools

import jax
import jax.numpy as jnp
from jax import lax
from jax.experimental import pallas as pl
from jax.experimental.pallas import tpu as pltpu
from jax.experimental.pallas import tpu_sc as plsc

D = 128
N_CAT = 26
D_CONT = D - N_CAT
VOCAB = 1000
EMB_DIM = 32
BATCH = 4096
OUT_D = D_CONT + N_CAT * EMB_DIM

NUM_CORES = 2
NUM_SUBCORES = 16
NW = NUM_CORES * NUM_SUBCORES
ROWS = BATCH // NW
CHUNK = 64
LANES = 16

CAT_BASE = 96
CAT_OFF = D_CONT - CAT_BASE


def _body(x_hbm, tab_hbm, out_hbm, obuf, gath, xcat, idx2d, sem_g, sem_c):
    wid = lax.axis_index("s") * NUM_CORES + lax.axis_index("c")
    lanes = lax.iota(jnp.int32, LANES)

    def chunk_pass(c, carry):
        base = wid * ROWS + c * CHUNK

        cont_cp = pltpu.make_async_copy(
            x_hbm.at[pl.ds(base, CHUNK), pl.ds(0, D_CONT + 2)],
            obuf.at[:, pl.ds(0, D_CONT + 2)],
            sem_c,
        )
        cont_cp.start()

        pltpu.sync_copy(
            x_hbm.at[pl.ds(base, CHUNK), pl.ds(CAT_BASE, 32)], xcat
        )

        def feat(i, cc):
            col = jnp.full((LANES,), i + CAT_OFF, jnp.int32)
            for m in range(CHUNK // LANES):
                rows = m * LANES + lanes
                v = plsc.load_gather(xcat, [rows, col])
                idx2d[i, pl.ds(m * LANES, LANES)] = v.astype(jnp.int32)
            pltpu.make_async_copy(
                tab_hbm.at[i].at[idx2d.at[i]], gath.at[i], sem_g
            ).start()
            return cc

        lax.fori_loop(0, N_CAT, feat, 0)

        cont_cp.wait()

        def drain(g, cc):
            pltpu.make_async_copy(
                tab_hbm.at[g].at[idx2d.at[g]], gath.at[g], sem_g
            ).wait()
            return cc

        lax.fori_loop(0, N_CAT, drain, 0)

        def place_row(r, cc):
            for g in range(N_CAT):
                dst = D_CONT + g * EMB_DIM
                lo = gath[g, r, pl.ds(0, LANES)]
                hi = gath[g, r, pl.ds(LANES, LANES)]
                obuf[r, pl.ds(dst, LANES)] = lo
                obuf[r, pl.ds(dst + LANES, LANES)] = hi
            return cc

        lax.fori_loop(0, CHUNK, place_row, 0)

        pltpu.sync_copy(obuf, out_hbm.at[pl.ds(base, CHUNK)])
        return carry

    lax.fori_loop(0, ROWS // CHUNK, chunk_pass, 0)


_sc_call = functools.partial(
    pl.kernel,
    mesh=plsc.VectorSubcoreMesh(core_axis_name="c", subcore_axis_name="s"),
    out_type=jax.ShapeDtypeStruct((BATCH, OUT_D), jnp.float32),
    scratch_types=[
        pltpu.VMEM((CHUNK, OUT_D), jnp.float32),
        pltpu.VMEM((N_CAT, CHUNK, EMB_DIM), jnp.float32),
        pltpu.VMEM((CHUNK, 32), jnp.float32),
        pltpu.VMEM((N_CAT, CHUNK), jnp.int32),
        pltpu.SemaphoreType.DMA,
        pltpu.SemaphoreType.DMA,
    ],
    compiler_params=pltpu.CompilerParams(
        use_tc_tiling_on_sc=False, needs_layout_passes=False
    ),
)(_body)


@jax.jit
def kernel(X, emb_tables, categ_idcs, non_categ_mask):
    return _sc_call(X, emb_tables)


# speedup vs baseline: 1.1120x; 1.0623x over previous
import functools

import jax
import jax.numpy as jnp
from jax import lax
from jax.experimental import pallas as pl
from jax.experimental.pallas import tpu as pltpu
from jax.experimental.pallas import tpu_sc as plsc

D = 128
N_CAT = 26
D_CONT = D - N_CAT
VOCAB = 1000
EMB_DIM = 32
BATCH = 4096
EMB_W = N_CAT * EMB_DIM

NUM_CORES = 2
NUM_SUBCORES = 16
NW = NUM_CORES * NUM_SUBCORES
ROWS = BATCH // NW
LANES = 16

CAT_BASE = 96
CAT_OFF = D_CONT - CAT_BASE


def _body(x_hbm, tab_hbm, emb_hbm, gath, xcat, idx2d, sem_g, sem_w):
    wid = lax.axis_index("s") * NUM_CORES + lax.axis_index("c")
    base = wid * ROWS
    lanes = lax.iota(jnp.int32, LANES)

    pltpu.sync_copy(x_hbm.at[pl.ds(base, ROWS), pl.ds(CAT_BASE, 32)], xcat)

    def feat(i, cc):
        col = jnp.full((LANES,), i + CAT_OFF, jnp.int32)
        for m in range(ROWS // LANES):
            rows = m * LANES + lanes
            v = plsc.load_gather(xcat, [rows, col])
            idx2d[i, pl.ds(m * LANES, LANES)] = v.astype(jnp.int32)
        pltpu.make_async_copy(
            tab_hbm.at[i].at[idx2d.at[i]], gath.at[i], sem_g
        ).start()
        return cc

    lax.fori_loop(0, N_CAT, feat, 0)

    def drain(g, cc):
        pltpu.make_async_copy(
            tab_hbm.at[g].at[idx2d.at[g]], gath.at[g], sem_g
        ).wait()
        return cc

    lax.fori_loop(0, N_CAT, drain, 0)

    def fire_out(g, cc):
        pltpu.make_async_copy(
            gath.at[g],
            emb_hbm.at[pl.ds(base, ROWS), pl.ds(g * EMB_DIM, EMB_DIM)],
            sem_w,
        ).start()
        return cc

    lax.fori_loop(0, N_CAT, fire_out, 0)

    def drain_out(g, cc):
        pltpu.make_async_copy(
            gath.at[g],
            emb_hbm.at[pl.ds(base, ROWS), pl.ds(g * EMB_DIM, EMB_DIM)],
            sem_w,
        ).wait()
        return cc

    lax.fori_loop(0, N_CAT, drain_out, 0)


_sc_call = functools.partial(
    pl.kernel,
    mesh=plsc.VectorSubcoreMesh(core_axis_name="c", subcore_axis_name="s"),
    out_type=jax.ShapeDtypeStruct((BATCH, EMB_W), jnp.float32),
    scratch_types=[
        pltpu.VMEM((N_CAT, ROWS, EMB_DIM), jnp.float32),
        pltpu.VMEM((ROWS, 32), jnp.float32),
        pltpu.VMEM((N_CAT, ROWS), jnp.int32),
        pltpu.SemaphoreType.DMA,
        pltpu.SemaphoreType.DMA,
    ],
    compiler_params=pltpu.CompilerParams(
        use_tc_tiling_on_sc=False, needs_layout_passes=False
    ),
)(_body)


@jax.jit
def kernel(X, emb_tables, categ_idcs, non_categ_mask):
    emb = _sc_call(X, emb_tables)
    return jnp.concatenate([X[:, :D_CONT], emb], axis=-1)
